# baseline (speedup 1.0000x reference)
import functools

import jax
import jax.numpy as jnp
from jax import lax
from jax.experimental import pallas as pl
from jax.experimental.pallas import tpu as pltpu
from jax.experimental.pallas import tpu_sc as plsc

NUM_CORES = 2
NUM_SUBCORES = 16
NUM_WORKERS = NUM_CORES * NUM_SUBCORES
LANES = 16
CHUNK = 64


def _compact_body(lo_ref, hi_ref, out_ref):
    out_ref[...] = jnp.concatenate([lo_ref[...], hi_ref[...]], axis=1)


def _compact(table):
    E, D = table.shape
    BR = 4096
    nblk = E // 2 // BR
    return pl.pallas_call(
        _compact_body,
        grid=(nblk,),
        in_specs=[
            pl.BlockSpec((BR, D), lambda i: (i, 0)),
            pl.BlockSpec((BR, D), lambda i: (i + nblk, 0)),
        ],
        out_specs=pl.BlockSpec((BR, 2 * D), lambda i: (i, 0)),
        out_shape=jax.ShapeDtypeStruct((E // 2, 2 * D), jnp.float32),
    )(table, table)


def kernel(h, r, t, entity_table, relation_table):
    batch = h.shape[0]
    ecount, dim = entity_table.shape
    assert batch % (8 * NUM_WORKERS) == 0 and dim == 64
    b_per_w = batch // NUM_WORKERS
    n_chunks = b_per_w // CHUNK

    etab2 = _compact(entity_table)
    rtab2 = _compact(relation_table)
    half = ecount // 2

    mesh = plsc.VectorSubcoreMesh(core_axis_name="c", subcore_axis_name="s")
    out_sds = jax.ShapeDtypeStruct((batch, dim), jnp.float32)

    @functools.partial(
        pl.kernel,
        out_type=(out_sds, out_sds, out_sds),
        mesh=mesh,
        compiler_params=pltpu.CompilerParams(needs_layout_passes=False),
        scratch_types=[
            pltpu.VMEM((b_per_w,), jnp.int32),
            pltpu.VMEM((b_per_w,), jnp.int32),
            pltpu.VMEM((CHUNK, 2 * dim), jnp.float32),
            pltpu.VMEM((CHUNK, dim), jnp.float32),
            pltpu.SemaphoreType.DMA,
        ],
    )
    def emb_kernel(h_hbm, r_hbm, t_hbm, etab, rtab, h_out, r_out, t_out,
                   idx_v, pid_v, packed_v, rows_v, sem):
        wid = lax.axis_index("s") * NUM_CORES + lax.axis_index("c")
        base = wid * b_per_w
        lane = lax.iota(jnp.int32, LANES)

        def run_lookup(src_hbm, table, out_hbm):
            pltpu.sync_copy(src_hbm.at[pl.ds(base, b_per_w)], idx_v)

            def pid_body(k, _):
                off = pl.ds(k * LANES, LANES)
                iv = idx_v[off]
                hi = lax.convert_element_type(iv >= half, jnp.int32)
                pid_v[off] = iv - hi * half
                return 0
            lax.fori_loop(0, b_per_w // LANES, pid_body, 0)

            def chunk_body(c, _):
                cb = c * CHUNK
                pltpu.async_copy(
                    table.at[pid_v.at[pl.ds(cb, CHUNK)]], packed_v, sem
                ).wait()
                for g in range(CHUNK // LANES):
                    jv = lane + g * LANES
                    ivec = idx_v[pl.ds(cb + g * LANES, LANES)]
                    hi = lax.convert_element_type(ivec >= half, jnp.int32)
                    bv = lax.shift_left(hi, 6)
                    def col_body(q, _):
                        cv = jnp.full((LANES,), q, jnp.int32)
                        vals = plsc.load_gather(packed_v, [jv, bv + cv])
                        plsc.store_scatter(rows_v, [jv, cv], vals)
                        return 0
                    lax.fori_loop(0, dim, col_body, 0)
                pltpu.sync_copy(rows_v, out_hbm.at[pl.ds(base + cb, CHUNK)])
                return 0
            lax.fori_loop(0, n_chunks, chunk_body, 0)

        run_lookup(h_hbm, etab, h_out)
        run_lookup(r_hbm, rtab, r_out)
        run_lookup(t_hbm, etab, t_out)

    return emb_kernel(h, r, t, etab2, rtab2)


# speedup vs baseline: 1.0071x; 1.0071x over previous
import functools

import jax
import jax.numpy as jnp
from jax import lax
from jax.experimental import pallas as pl
from jax.experimental.pallas import tpu as pltpu
from jax.experimental.pallas import tpu_sc as plsc

NUM_CORES = 2
NUM_SUBCORES = 16
NUM_WORKERS = NUM_CORES * NUM_SUBCORES
LANES = 16
CHUNK = 64


def _compact_body(lo_ref, hi_ref, out_ref):
    out_ref[...] = jnp.concatenate([lo_ref[...], hi_ref[...]], axis=1)


def _compact(table):
    E, D = table.shape
    BR = 5000
    nblk = E // 2 // BR
    return pl.pallas_call(
        _compact_body,
        grid=(nblk,),
        in_specs=[
            pl.BlockSpec((BR, D), lambda i: (i, 0)),
            pl.BlockSpec((BR, D), lambda i: (i + nblk, 0)),
        ],
        out_specs=pl.BlockSpec((BR, 2 * D), lambda i: (i, 0)),
        out_shape=jax.ShapeDtypeStruct((E // 2, 2 * D), jnp.float32),
    )(table, table)


def kernel(h, r, t, entity_table, relation_table):
    batch = h.shape[0]
    ecount, dim = entity_table.shape
    assert batch % (8 * NUM_WORKERS) == 0 and dim == 64
    b_per_w = batch // NUM_WORKERS
    n_chunks = b_per_w // CHUNK

    etab2 = _compact(entity_table)
    rtab2 = _compact(relation_table)
    half = ecount // 2

    mesh = plsc.VectorSubcoreMesh(core_axis_name="c", subcore_axis_name="s")
    out_sds = jax.ShapeDtypeStruct((batch, dim), jnp.float32)

    @functools.partial(
        pl.kernel,
        out_type=(out_sds, out_sds, out_sds),
        mesh=mesh,
        compiler_params=pltpu.CompilerParams(needs_layout_passes=False),
        scratch_types=[
            pltpu.VMEM((b_per_w,), jnp.int32),
            pltpu.VMEM((b_per_w,), jnp.int32),
            pltpu.VMEM((CHUNK, 2 * dim), jnp.float32),
            pltpu.VMEM((CHUNK, dim), jnp.float32),
            pltpu.SemaphoreType.DMA,
        ],
    )
    def emb_kernel(h_hbm, r_hbm, t_hbm, etab, rtab, h_out, r_out, t_out,
                   idx_v, pid_v, packed_v, rows_v, sem):
        wid = lax.axis_index("s") * NUM_CORES + lax.axis_index("c")
        base = wid * b_per_w
        lane = lax.iota(jnp.int32, LANES)

        def run_lookup(src_hbm, table, out_hbm):
            pltpu.sync_copy(src_hbm.at[pl.ds(base, b_per_w)], idx_v)

            def pid_body(k, _):
                off = pl.ds(k * LANES, LANES)
                iv = idx_v[off]
                hi = lax.convert_element_type(iv >= half, jnp.int32)
                pid_v[off] = iv - hi * half
                return 0
            lax.fori_loop(0, b_per_w // LANES, pid_body, 0)

            def chunk_body(c, _):
                cb = c * CHUNK
                pltpu.async_copy(
                    table.at[pid_v.at[pl.ds(cb, CHUNK)]], packed_v, sem
                ).wait()
                for g in range(CHUNK // LANES):
                    jv = lane + g * LANES
                    ivec = idx_v[pl.ds(cb + g * LANES, LANES)]
                    hi = lax.convert_element_type(ivec >= half, jnp.int32)
                    bv = lax.shift_left(hi, 6)
                    def col_body(q, _):
                        cv = jnp.full((LANES,), q, jnp.int32)
                        vals = plsc.load_gather(packed_v, [jv, bv + cv])
                        plsc.store_scatter(rows_v, [jv, cv], vals)
                        return 0
                    lax.fori_loop(0, dim, col_body, 0)
                pltpu.sync_copy(rows_v, out_hbm.at[pl.ds(base + cb, CHUNK)])
                return 0
            lax.fori_loop(0, n_chunks, chunk_body, 0)

        run_lookup(h_hbm, etab, h_out)
        run_lookup(r_hbm, rtab, r_out)
        run_lookup(t_hbm, etab, t_out)

    return emb_kernel(h, r, t, etab2, rtab2)


# speedup vs baseline: 1.0169x; 1.0098x over previous
import functools

import jax
import jax.numpy as jnp
from jax import lax
from jax.experimental import pallas as pl
from jax.experimental.pallas import tpu as pltpu
from jax.experimental.pallas import tpu_sc as plsc

NUM_CORES = 2
NUM_SUBCORES = 16
NUM_WORKERS = NUM_CORES * NUM_SUBCORES
LANES = 16
CHUNK = 64


def _compact_body(lo_ref, hi_ref, out_ref):
    out_ref[...] = jnp.concatenate([lo_ref[...], hi_ref[...]], axis=1)


def _compact(table):
    E, D = table.shape
    BR = 10000
    nblk = E // 2 // BR
    return pl.pallas_call(
        _compact_body,
        grid=(nblk,),
        in_specs=[
            pl.BlockSpec((BR, D), lambda i: (i, 0)),
            pl.BlockSpec((BR, D), lambda i: (i + nblk, 0)),
        ],
        out_specs=pl.BlockSpec((BR, 2 * D), lambda i: (i, 0)),
        out_shape=jax.ShapeDtypeStruct((E // 2, 2 * D), jnp.float32),
    )(table, table)


def kernel(h, r, t, entity_table, relation_table):
    batch = h.shape[0]
    ecount, dim = entity_table.shape
    assert batch % (8 * NUM_WORKERS) == 0 and dim == 64
    b_per_w = batch // NUM_WORKERS
    n_chunks = b_per_w // CHUNK

    etab2 = _compact(entity_table)
    rtab2 = _compact(relation_table)
    half = ecount // 2

    mesh = plsc.VectorSubcoreMesh(core_axis_name="c", subcore_axis_name="s")
    out_sds = jax.ShapeDtypeStruct((batch, dim), jnp.float32)

    @functools.partial(
        pl.kernel,
        out_type=(out_sds, out_sds, out_sds),
        mesh=mesh,
        compiler_params=pltpu.CompilerParams(needs_layout_passes=False),
        scratch_types=[
            pltpu.VMEM((b_per_w,), jnp.int32),
            pltpu.VMEM((b_per_w,), jnp.int32),
            pltpu.VMEM((CHUNK, 2 * dim), jnp.float32),
            pltpu.VMEM((CHUNK, dim), jnp.float32),
            pltpu.SemaphoreType.DMA,
        ],
    )
    def emb_kernel(h_hbm, r_hbm, t_hbm, etab, rtab, h_out, r_out, t_out,
                   idx_v, pid_v, packed_v, rows_v, sem):
        wid = lax.axis_index("s") * NUM_CORES + lax.axis_index("c")
        base = wid * b_per_w
        lane = lax.iota(jnp.int32, LANES)

        def run_lookup(src_hbm, table, out_hbm):
            pltpu.sync_copy(src_hbm.at[pl.ds(base, b_per_w)], idx_v)

            def pid_body(k, _):
                off = pl.ds(k * LANES, LANES)
                iv = idx_v[off]
                hi = lax.convert_element_type(iv >= half, jnp.int32)
                pid_v[off] = iv - hi * half
                return 0
            lax.fori_loop(0, b_per_w // LANES, pid_body, 0)

            def chunk_body(c, _):
                cb = c * CHUNK
                pltpu.async_copy(
                    table.at[pid_v.at[pl.ds(cb, CHUNK)]], packed_v, sem
                ).wait()
                for g in range(CHUNK // LANES):
                    jv = lane + g * LANES
                    ivec = idx_v[pl.ds(cb + g * LANES, LANES)]
                    hi = lax.convert_element_type(ivec >= half, jnp.int32)
                    bv = lax.shift_left(hi, 6)
                    def col_body(q, _):
                        cv = jnp.full((LANES,), q, jnp.int32)
                        vals = plsc.load_gather(packed_v, [jv, bv + cv])
                        plsc.store_scatter(rows_v, [jv, cv], vals)
                        return 0
                    lax.fori_loop(0, dim, col_body, 0)
                pltpu.sync_copy(rows_v, out_hbm.at[pl.ds(base + cb, CHUNK)])
                return 0
            lax.fori_loop(0, n_chunks, chunk_body, 0)

        run_lookup(h_hbm, etab, h_out)
        run_lookup(r_hbm, rtab, r_out)
        run_lookup(t_hbm, etab, t_out)

    return emb_kernel(h, r, t, etab2, rtab2)


# speedup vs baseline: 1.8153x; 1.7850x over previous
import functools

import jax
import jax.numpy as jnp
from jax import lax
from jax.experimental import pallas as pl
from jax.experimental.pallas import tpu as pltpu
from jax.experimental.pallas import tpu_sc as plsc

NUM_CORES = 2
NUM_SUBCORES = 16
NUM_WORKERS = NUM_CORES * NUM_SUBCORES
CHUNK = 64
NBUF = 2


def kernel(h, r, t, entity_table, relation_table):
    batch = h.shape[0]
    dim = entity_table.shape[1]
    assert batch % (8 * NUM_WORKERS) == 0
    b_per_w = batch // NUM_WORKERS
    n_chunks = b_per_w // CHUNK
    assert n_chunks % NBUF == 0

    mesh = plsc.VectorSubcoreMesh(core_axis_name="c", subcore_axis_name="s")
    out_sds = jax.ShapeDtypeStruct((batch, dim), jnp.float32)

    @functools.partial(
        pl.kernel,
        out_type=(out_sds, out_sds, out_sds),
        mesh=mesh,
        scratch_types=[
            pltpu.VMEM((b_per_w,), jnp.int32),
            pltpu.VMEM((NBUF, CHUNK, dim), jnp.float32),
            pltpu.SemaphoreType.DMA,
            pltpu.SemaphoreType.DMA,
        ],
    )
    def emb_kernel(h_hbm, r_hbm, t_hbm, etab, rtab, h_out, r_out, t_out,
                   idx_v, rows_v, sem, store_sem):
        wid = lax.axis_index("s") * NUM_CORES + lax.axis_index("c")
        base = wid * b_per_w

        def run_lookup(li, src_hbm, table, out_hbm):
            pltpu.sync_copy(src_hbm.at[pl.ds(base, b_per_w)], idx_v)

            def pair_body(i, _):
                cc = i * NBUF
                for b in range(NBUF):
                    cb = (cc + b) * CHUNK
                    copies = []
                    for g in range(CHUNK // 16):
                        ivec = idx_v[pl.ds(cb + g * 16, 16)]
                        for k in range(16):
                            copies.append(pltpu.async_copy(
                                table.at[ivec[k]],
                                rows_v.at[b, g * 16 + k], sem))

                    def drain_store():
                        pltpu.make_async_copy(
                            rows_v.at[b], out_hbm.at[pl.ds(base, CHUNK)],
                            store_sem).wait()
                    if li == 0:
                        @pl.when(cc >= NBUF)
                        def _():
                            drain_store()
                    else:
                        drain_store()

                    for cp in copies:
                        cp.wait()
                    pltpu.async_copy(
                        rows_v.at[b], out_hbm.at[pl.ds(base + cb, CHUNK)],
                        store_sem)
                return 0
            lax.fori_loop(0, n_chunks // NBUF, pair_body, 0)

        run_lookup(0, h_hbm, etab, h_out)
        run_lookup(1, r_hbm, rtab, r_out)
        run_lookup(2, t_hbm, etab, t_out)

        for b in range(NBUF):
            pltpu.make_async_copy(
                rows_v.at[b], t_out.at[pl.ds(base, CHUNK)], store_sem).wait()

    return emb_kernel(h, r, t, entity_table, relation_table)
